# Initial kernel scaffold; baseline (speedup 1.0000x reference)
import functools

import jax
import jax.numpy as jnp
from jax import lax
from jax.experimental import pallas as pl
from jax.experimental.pallas import tpu as pltpu
from jax.experimental.pallas import tpu_sc as plsc

N = 10000
E = 320000
D = 128
H = 128
G = 64

WD = H + 8
CH = 128
EC = E // CH
NC = 2
NS = 16
NW = NC * NS
RPT = N // NS
BM = 2000
NB = N // BM

_f32 = jnp.float32


def _sc_agg_body(src_hbm, dst_hbm, z_hbm, zz_hbm, out_hbm,
                 src_v, dst_v, rows_v, acc_sh, sem):
    c = lax.axis_index("c")
    s = lax.axis_index("s")
    w = s * NC + c
    r0 = s * RPT

    pltpu.sync_copy(zz_hbm.at[pl.ds(r0, RPT)], acc_sh.at[pl.ds(r0, RPT)])
    plsc.subcore_barrier()

    steps = (EC + NW - 1) // NW

    def body(j, carry):
        cid = w + j * NW

        @pl.when(cid < EC)
        def _():
            pltpu.sync_copy(src_hbm.at[cid], src_v)
            pltpu.sync_copy(dst_hbm.at[cid], dst_v)
            pltpu.async_copy(z_hbm.at[src_v], rows_v, sem).wait()
            pltpu.sync_copy(rows_v, acc_sh.at[dst_v], add=True)

        return carry

    lax.fori_loop(0, steps, body, 0)
    plsc.subcore_barrier()

    pltpu.sync_copy(acc_sh.at[pl.ds(r0, RPT)],
                    out_hbm.at[c, pl.ds(r0, RPT)])


def _make_sc_agg():
    mesh = plsc.VectorSubcoreMesh(core_axis_name="c", subcore_axis_name="s")
    return functools.partial(
        pl.kernel,
        mesh=mesh,
        out_type=jax.ShapeDtypeStruct((NC, N, WD), _f32),
        scratch_types=[
            pltpu.VMEM((CH,), jnp.int32),
            pltpu.VMEM((CH,), jnp.int32),
            pltpu.VMEM((CH, WD), _f32),
            pltpu.VMEM_SHARED((N, WD), _f32),
            pltpu.SemaphoreType.DMA,
        ],
    )(_sc_agg_body)


def _ones_pad(m):
    ex = (lax.broadcasted_iota(jnp.int32, (m.shape[0], WD - H), 1) == 0)
    return jnp.concatenate([m, ex.astype(_f32)], axis=1)


def _pre_body(x_ref, wl_ref, z_ref):
    z_ref[...] = _ones_pad(
        jnp.dot(x_ref[...], wl_ref[...], preferred_element_type=_f32))


def _mid_body(p_ref, xp_ref, wr_ref, bl_ref, wln_ref, h_ref, z_ref):
    p = p_ref[0] + p_ref[1]
    inv = 1.0 / jnp.maximum(p[:, H:H + 1], 1.0)
    agg = p[:, :H] * inv
    h = jnp.maximum(
        agg + bl_ref[...] +
        jnp.dot(xp_ref[...], wr_ref[...], preferred_element_type=_f32), 0.0)
    h_ref[...] = h
    z_ref[...] = _ones_pad(
        jnp.dot(h, wln_ref[...], preferred_element_type=_f32))


def _mid3_body(p_ref, xp_ref, wr_ref, bl_ref, h_ref):
    p = p_ref[0] + p_ref[1]
    inv = 1.0 / jnp.maximum(p[:, H:H + 1], 1.0)
    agg = p[:, :H] * inv
    h_ref[...] = jnp.maximum(
        agg + bl_ref[...] +
        jnp.dot(xp_ref[...], wr_ref[...], preferred_element_type=_f32), 0.0)


def _pool_body(h_ref, b_ref, wgt_ref, bg_ref, w1_ref, b1_ref, w2t_ref,
               b2_ref, o_ref):
    h = h_ref[...]
    gate = jnp.sum(h * wgt_ref[...], axis=1, keepdims=True) + bg_ref[...]
    mask = b_ref[...] == lax.broadcasted_iota(jnp.int32, (N, G), 1)
    mg = jnp.max(jnp.where(mask, gate, -3e38), axis=0, keepdims=True)
    ew = jnp.where(mask, jnp.exp(gate - mg), 0.0)
    dn = jnp.sum(ew, axis=0, keepdims=True)
    alpha = ew / jnp.where(dn > 0.0, dn, 1.0)
    pooled = lax.dot_general(alpha, h, (((0,), (0,)), ((), ())),
                             preferred_element_type=_f32)
    t = jnp.maximum(
        jnp.dot(pooled, w1_ref[...], preferred_element_type=_f32) +
        b1_ref[...], 0.0)
    o_ref[...] = jnp.sum(t * w2t_ref[...], axis=1, keepdims=True) + b2_ref[...]


def _row_spec(width):
    return pl.BlockSpec((BM, width), lambda i: (i, 0))


def _rep_spec(shape):
    nd = len(shape)
    return pl.BlockSpec(shape, lambda i: (0,) * nd)


def _tc_pre(x, wl):
    return pl.pallas_call(
        _pre_body,
        grid=(NB,),
        in_specs=[_row_spec(D), _rep_spec((D, H))],
        out_specs=_row_spec(WD),
        out_shape=jax.ShapeDtypeStruct((N, WD), _f32),
    )(x, wl)


def _tc_mid(p, xp, wr, bl, wln):
    return pl.pallas_call(
        _mid_body,
        grid=(NB,),
        in_specs=[
            pl.BlockSpec((NC, BM, WD), lambda i: (0, i, 0)),
            _row_spec(H), _rep_spec((H, H)), _rep_spec((1, H)),
            _rep_spec((H, H)),
        ],
        out_specs=[_row_spec(H), _row_spec(WD)],
        out_shape=[jax.ShapeDtypeStruct((N, H), _f32),
                   jax.ShapeDtypeStruct((N, WD), _f32)],
    )(p, xp, wr, bl, wln)


def _tc_mid3(p, xp, wr, bl):
    return pl.pallas_call(
        _mid3_body,
        grid=(NB,),
        in_specs=[
            pl.BlockSpec((NC, BM, WD), lambda i: (0, i, 0)),
            _row_spec(H), _rep_spec((H, H)), _rep_spec((1, H)),
        ],
        out_specs=_row_spec(H),
        out_shape=jax.ShapeDtypeStruct((N, H), _f32),
    )(p, xp, wr, bl)


def _tc_pool(h, batch2d, wgt, bg, w1, b1, w2t, b2):
    return pl.pallas_call(
        _pool_body,
        out_shape=jax.ShapeDtypeStruct((G, 1), _f32),
    )(h, batch2d, wgt, bg, w1, b1, w2t, b2)


def kernel(x, edge_index, batch, Wl1, bl1, Wr1, Wl2, bl2, Wr2, Wl3, bl3, Wr3,
           Wg, bg, W1, b1, W2, b2):
    src2d = edge_index[0].reshape(EC, CH)
    dst2d = edge_index[1].reshape(EC, CH)
    batch2d = batch.reshape(N, 1)
    zz = jnp.zeros((N, WD), _f32)

    sc_agg = _make_sc_agg()

    z1 = _tc_pre(x, Wl1)
    p1 = sc_agg(src2d, dst2d, z1, zz)
    h1, z2 = _tc_mid(p1, x, Wr1, bl1.reshape(1, H), Wl2)
    p2 = sc_agg(src2d, dst2d, z2, zz)
    h2, z3 = _tc_mid(p2, h1, Wr2, bl2.reshape(1, H), Wl3)
    p3 = sc_agg(src2d, dst2d, z3, zz)
    h3 = _tc_mid3(p3, h2, Wr3, bl3.reshape(1, H))
    out = _tc_pool(h3, batch2d, Wg.T, bg.reshape(1, 1),
                   W1, b1.reshape(1, H), W2.T, b2.reshape(1, 1))
    return out


# speedup vs baseline: 5.5476x; 5.5476x over previous
import functools

import jax
import jax.numpy as jnp
from jax import lax
from jax.experimental import pallas as pl
from jax.experimental.pallas import tpu as pltpu
from jax.experimental.pallas import tpu_sc as plsc

N = 10000
E = 320000
D = 128
H = 128
G = 64

CH = 128
EC = E // CH
NC = 2
NS = 16
NW = NC * NS
NP = 10240
RPT = NP // NS
EPT = E // NS
BM = 2000
NB = N // BM

_f32 = jnp.float32


def _sc_mesh():
    return plsc.VectorSubcoreMesh(core_axis_name="c", subcore_axis_name="s")


def _sc_agg_body(src_hbm, dst_hbm, z_hbm, zz_hbm, out_hbm,
                 src_v, dst_v, rows_v, acc_sh, sem):
    c = lax.axis_index("c")
    s = lax.axis_index("s")
    w = s * NC + c
    r0 = s * RPT

    pltpu.sync_copy(zz_hbm.at[pl.ds(r0, RPT)], acc_sh.at[pl.ds(r0, RPT)])
    plsc.subcore_barrier()

    steps = (EC + NW - 1) // NW

    def body(j, carry):
        cid = w + j * NW

        @pl.when(cid < EC)
        def _():
            pltpu.sync_copy(src_hbm.at[cid], src_v)
            pltpu.sync_copy(dst_hbm.at[cid], dst_v)
            pltpu.async_copy(z_hbm.at[src_v], rows_v, sem).wait()
            pltpu.sync_copy(rows_v, acc_sh.at[dst_v], add=True)

        return carry

    lax.fori_loop(0, steps, body, 0)
    plsc.subcore_barrier()

    pltpu.sync_copy(acc_sh.at[pl.ds(r0, RPT)],
                    out_hbm.at[c, pl.ds(r0, RPT)])


def _make_sc_agg():
    return functools.partial(
        pl.kernel,
        mesh=_sc_mesh(),
        out_type=jax.ShapeDtypeStruct((NC, NP, H), _f32),
        scratch_types=[
            pltpu.VMEM((CH,), jnp.int32),
            pltpu.VMEM((CH,), jnp.int32),
            pltpu.VMEM((CH, H), _f32),
            pltpu.VMEM_SHARED((NP, H), _f32),
            pltpu.SemaphoreType.DMA,
        ],
    )(_sc_agg_body)


def _sc_cnt_body(dst_hbm, ones_hbm, zz_hbm, out_hbm,
                 dst_v, ones_v, acc_sh, sem):
    c = lax.axis_index("c")
    s = lax.axis_index("s")
    w = s * NC + c
    r0 = s * RPT

    pltpu.sync_copy(zz_hbm.at[pl.ds(r0, RPT)], acc_sh.at[pl.ds(r0, RPT)])
    pltpu.sync_copy(ones_hbm, ones_v)
    plsc.subcore_barrier()

    steps = (EC + NW - 1) // NW

    def body(j, carry):
        cid = w + j * NW

        @pl.when(cid < EC)
        def _():
            pltpu.sync_copy(dst_hbm.at[cid], dst_v)
            pltpu.sync_copy(ones_v, acc_sh.at[dst_v], add=True)

        return carry

    lax.fori_loop(0, steps, body, 0)
    plsc.subcore_barrier()

    pltpu.sync_copy(acc_sh.at[pl.ds(r0, RPT)],
                    out_hbm.at[c, pl.ds(r0, RPT)])


def _make_sc_cnt():
    return functools.partial(
        pl.kernel,
        mesh=_sc_mesh(),
        out_type=jax.ShapeDtypeStruct((NC, NP, H), _f32),
        scratch_types=[
            pltpu.VMEM((CH,), jnp.int32),
            pltpu.VMEM((CH, H), _f32),
            pltpu.VMEM_SHARED((NP, H), _f32),
            pltpu.SemaphoreType.DMA,
        ],
    )(_sc_cnt_body)


def _recip(c):
    r = 1.0 / c
    r = r * (2.0 - c * r)
    r = r * (2.0 - c * r)
    return r


def _layer_body(p_ref, cnt_ref, xp_ref, wl_ref, bl_ref, wr_ref, h_ref):
    inv = _recip(jnp.maximum(cnt_ref[0] + cnt_ref[1], 1.0))
    mean = (p_ref[0] + p_ref[1]) * inv
    h_ref[...] = jnp.maximum(
        jnp.dot(mean, wl_ref[...], preferred_element_type=_f32) +
        bl_ref[...] +
        jnp.dot(xp_ref[...], wr_ref[...], preferred_element_type=_f32), 0.0)


def _pool_body(h_ref, b_ref, wg_ref, bg_ref, w1_ref, b1_ref, w2_ref,
               b2_ref, o_ref):
    h = h_ref[...]
    gate = jnp.dot(h, wg_ref[...], preferred_element_type=_f32) + bg_ref[...]
    mask = b_ref[...] == lax.broadcasted_iota(jnp.int32, (N, G), 1)
    mg = jnp.max(jnp.where(mask, gate, -3e38), axis=0, keepdims=True)
    ew = jnp.where(mask, jnp.exp(gate - mg), 0.0)
    dn = jnp.sum(ew, axis=0, keepdims=True)
    alpha = ew * _recip(jnp.where(dn > 0.0, dn, 1.0))
    pooled = lax.dot_general(alpha, h, (((0,), (0,)), ((), ())),
                             preferred_element_type=_f32,
                             precision=lax.Precision.HIGHEST)
    t = jnp.maximum(
        jnp.dot(pooled, w1_ref[...], preferred_element_type=_f32) +
        b1_ref[...], 0.0)
    o_ref[...] = jnp.dot(t, w2_ref[...], preferred_element_type=_f32) + b2_ref[...]


def _row_spec(width):
    return pl.BlockSpec((BM, width), lambda i: (i, 0))


def _rep_spec(shape):
    nd = len(shape)
    return pl.BlockSpec(shape, lambda i: (0,) * nd)


def _tc_layer(p, cnt2d, xp, wl, bl, wr):
    return pl.pallas_call(
        _layer_body,
        grid=(NB,),
        in_specs=[
            pl.BlockSpec((NC, BM, H), lambda i: (0, i, 0)),
            pl.BlockSpec((NC, BM, 1), lambda i: (0, i, 0)),
            _row_spec(H), _rep_spec((H, H)), _rep_spec((1, H)),
            _rep_spec((H, H)),
        ],
        out_specs=_row_spec(H),
        out_shape=jax.ShapeDtypeStruct((N, H), _f32),
    )(p, cnt2d, xp, wl, bl, wr)


def _tc_pool(h, batch2d, wg, bg, w1, b1, w2, b2):
    return pl.pallas_call(
        _pool_body,
        out_shape=jax.ShapeDtypeStruct((G, 1), _f32),
    )(h, batch2d, wg, bg, w1, b1, w2, b2)


def kernel(x, edge_index, batch, Wl1, bl1, Wr1, Wl2, bl2, Wr2, Wl3, bl3, Wr3,
           Wg, bg, W1, b1, W2, b2):
    src2d = edge_index[0].reshape(EC, CH)
    dst2d = edge_index[1].reshape(EC, CH)
    batch2d = batch.reshape(N, 1)
    zz = jnp.zeros((NP, H), _f32)

    sc_agg = _make_sc_agg()
    cnt2d = _make_sc_cnt()(dst2d, jnp.ones((CH, H), _f32), zz)[:, :, :1]

    p1 = sc_agg(src2d, dst2d, x, zz)
    h1 = _tc_layer(p1, cnt2d, x, Wl1, bl1.reshape(1, H), Wr1)
    p2 = sc_agg(src2d, dst2d, h1, zz)
    h2 = _tc_layer(p2, cnt2d, h1, Wl2, bl2.reshape(1, H), Wr2)
    p3 = sc_agg(src2d, dst2d, h2, zz)
    h3 = _tc_layer(p3, cnt2d, h2, Wl3, bl3.reshape(1, H), Wr3)
    out = _tc_pool(h3, batch2d, Wg, bg.reshape(1, 1),
                   W1, b1.reshape(1, H), W2, b2.reshape(1, 1))
    return out
